# baseline (speedup 1.0000x reference)
import functools

import jax
import jax.numpy as jnp
import numpy as np
from jax import lax
from jax.experimental import pallas as pl
from jax.experimental.pallas import tpu as pltpu
from jax.experimental.pallas import tpu_sc as plsc

LEVELS = 4
BATCH = 128
D = 1024
NPAIR = 8128
P_EXP = 2

NC = 2
NS = 16
NW = NC * NS
LANES = 16
BR = 16
NB = BATCH // BR
NOFF = (NB * (NB - 1)) // 2
SEC = 512
NSEC = D // SEC
MS = SEC // LANES

_mesh = plsc.VectorSubcoreMesh(core_axis_name="c", subcore_axis_name="s")


def _off(i):
    return i * (BATCH - 1) - (i * (i - 1)) // 2


@functools.partial(
    pl.kernel,
    mesh=_mesh,
    compiler_params=pltpu.CompilerParams(use_tc_tiling_on_sc=False),
    out_type=[jax.ShapeDtypeStruct((NPAIR, D), jnp.float32)
              for _ in range(LEVELS)],
    scratch_types=[
        pltpu.VMEM((BR, D), jnp.float32),
        pltpu.VMEM((BR, D), jnp.float32),
        pltpu.VMEM((BR, D), jnp.float32),
        pltpu.VMEM((BR, D), jnp.float32),
        pltpu.VMEM((BR, D), jnp.float32),
        pltpu.VMEM((BR, D), jnp.float32),
        pltpu.SemaphoreType.DMA,
        pltpu.SemaphoreType.DMA,
        pltpu.SemaphoreType.DMA,
        pltpu.SemaphoreType.DMA,
        pltpu.SemaphoreType.DMA,
        pltpu.SemaphoreType.DMA,
        pltpu.SemaphoreType.DMA,
    ],
)
def _batch_diff_sc(table_hbm, out0, out1, out2, out3,
                   rja0, rja1, rjb0, rjb1, ob0, ob1,
                   sra0, sra1, srb0, srb1, sw0, sw1, st):
    sid = lax.axis_index("s")
    cid = lax.axis_index("c")
    wid = sid * NC + cid
    outs = (out0, out1, out2, out3)
    rja = (rja0, rja1)
    rjb = (rjb0, rjb1)
    ob = (ob0, ob1)
    sra = (sra0, sra1)
    srb = (srb0, srb1)
    sw = (sw0, sw1)

    w = wid
    bi = ((w >= 7).astype(jnp.int32) + (w >= 13) + (w >= 18)
          + (w >= 22) + (w >= 25) + (w >= 27))
    bj = w - (bi * (2 * NB - 1 - bi)) // 2 + bi + 1
    d0 = 2 * (w - NOFF)

    ra_row = jnp.where(w < NOFF, bi, d0) * BR
    rb_row = jnp.where(w < NOFF, bj, d0 + 1) * BR

    def rd_issue(l, s):
        lbase = l * BATCH
        pltpu.async_copy(table_hbm.at[pl.ds(lbase + ra_row, BR)],
                         rja[s], sra[s])
        pltpu.async_copy(table_hbm.at[pl.ds(lbase + rb_row, BR)],
                         rjb[s], srb[s])

    def rd_wait(s):
        pltpu.make_async_copy(table_hbm.at[pl.ds(0, BR)], rja[s],
                              sra[s]).wait()
        pltpu.make_async_copy(table_hbm.at[pl.ds(0, BR)], rjb[s],
                              srb[s]).wait()

    rd_issue(0, 0)

    for l in range(LEVELS):
        out_l = outs[l]
        s = l % 2
        rd_wait(s)
        if l + 1 < LEVELS:
            rd_issue(l + 1, 1 - s)

        @pl.when(w < NOFF)
        def _(s=s, out_l=out_l):
            def row_pair(g, _):
                for b2 in range(2):
                    r = 2 * g + b2

                    @pl.when(r >= 2)
                    def _(b2=b2):
                        pltpu.make_async_copy(
                            ob[b2], out_l.at[pl.ds(0, BR)], sw[b2]).wait()

                    for sec in range(NSEC):
                        a_reg = [rja[s][r, pl.ds(sec * SEC + m * LANES,
                                                 LANES)]
                                 for m in range(MS)]

                        def col_body(j, carry, b2=b2, a_reg=a_reg, sec=sec):
                            for m in range(MS):
                                sl = pl.ds(sec * SEC + m * LANES, LANES)
                                dd = a_reg[m] - rjb[s][j, sl]
                                ob[b2][j, sl] = dd * dd
                            return carry

                        lax.fori_loop(0, BR, col_body, 0)

                    i = bi * BR + r
                    out_off = _off(i) + bj * BR - i - 1
                    pltpu.async_copy(ob[b2],
                                     out_l.at[pl.ds(out_off, BR)], sw[b2])
                return 0

            lax.fori_loop(0, BR // 2, row_pair, 0)
            for b2 in range(2):
                pltpu.make_async_copy(ob[b2], out_l.at[pl.ds(0, BR)],
                                      sw[b2]).wait()

        @pl.when(w >= NOFF)
        def _(s=s, out_l=out_l):
            for t in range(2):
                win = rja[s] if t == 0 else rjb[s]
                dband = d0 + t

                def tri_row(r, carry, win=win, dband=dband):
                    rlen = BR - 1 - r
                    i = dband * BR + r
                    obase = _off(i)

                    def tri_col(jj, carry2, win=win, r=r, obase=obase):
                        for m in range(D // LANES):
                            sl = pl.ds(m * LANES, LANES)
                            dd = win[r, sl] - win[r + 1 + jj, sl]
                            ob[0][jj, sl] = dd * dd
                        pltpu.async_copy(ob[0].at[pl.ds(jj, 1)],
                                         out_l.at[pl.ds(obase + jj, 1)], st)
                        return carry2

                    lax.fori_loop(0, rlen, tri_col, 0)

                    def tri_drain(jj, carry2):
                        pltpu.make_async_copy(
                            ob[0].at[pl.ds(0, 1)],
                            out_l.at[pl.ds(0, 1)], st).wait()
                        return carry2

                    lax.fori_loop(0, rlen, tri_drain, 0)
                    return carry

                lax.fori_loop(0, BR - 1, tri_row, 0)


def kernel(pyramid):
    table = pyramid.reshape(LEVELS * BATCH, D)
    return tuple(_batch_diff_sc(table))
